# baseline (speedup 1.0000x reference)
import functools

import jax
import jax.numpy as jnp
from jax import lax
from jax.experimental import pallas as pl
from jax.experimental.pallas import tpu as pltpu
from jax.experimental.pallas import tpu_sc as plsc

NC = 2
NS = 16
NW = NC * NS
CH = 64
LANES = 16


def _deg_kernel(np_, cpw, rpt):
    batch_k = 8

    def body(dst_hbm, ew_hbm, deg_hbm, dbuf, ebuf, zbuf, deg_sh, sem):
        cid = lax.axis_index("c")
        sid = lax.axis_index("s")
        wid = cid * NS + sid
        zv = jnp.zeros((LANES,), jnp.float32)

        def zero_buf(j, c):
            zbuf[pl.ds(j * LANES, LANES)] = zv
            return c
        lax.fori_loop(0, rpt // LANES, zero_buf, 0)
        pltpu.sync_copy(zbuf, deg_sh.at[pl.ds(sid * rpt, rpt)])
        pltpu.sync_copy(dst_hbm.at[wid], dbuf)
        pltpu.sync_copy(ew_hbm.at[wid], ebuf)
        plsc.subcore_barrier()

        def batch(t, c):
            for b in range(batch_k):
                j = t * batch_k + b
                pltpu.async_copy(ebuf.at[j], deg_sh.at[dbuf.at[j]], sem,
                                 add=True)
            for b in range(batch_k):
                j = t * batch_k + b
                pltpu.make_async_copy(ebuf.at[j], deg_sh.at[dbuf.at[j]],
                                      sem).wait()
            return c
        lax.fori_loop(0, cpw // batch_k, batch, 0)
        plsc.subcore_barrier()
        pltpu.sync_copy(deg_sh.at[pl.ds(sid * rpt, rpt)],
                        deg_hbm.at[cid, pl.ds(sid * rpt, rpt)])

    return pl.kernel(
        body,
        out_type=jax.ShapeDtypeStruct((NC, np_), jnp.float32),
        mesh=plsc.VectorSubcoreMesh(core_axis_name="c", subcore_axis_name="s"),
        scratch_types=[
            pltpu.VMEM((cpw, CH), jnp.int32),
            pltpu.VMEM((cpw, CH), jnp.float32),
            pltpu.VMEM((rpt,), jnp.float32),
            pltpu.VMEM_SHARED((np_,), jnp.float32),
            pltpu.SemaphoreType.DMA,
        ],
    )


def _scatter_kernel(np_, feat, cpw, rpt):
    def body(ebc_hbm, ewx_hbm, xs_hbm, s_hbm,
             ib0, ib1, ib2, ib3, eb0, eb1, eb2, eb3,
             g0, g1, v0, v1, s_sh,
             is0, is1, is2, is3, gs0, gs1, ss0, ss1):
        cid = lax.axis_index("c")
        sid = lax.axis_index("s")
        wid = cid * NS + sid
        ibs = (ib0, ib1, ib2, ib3)
        ebs = (eb0, eb1, eb2, eb3)
        iss = (is0, is1, is2, is3)
        gbs = (g0, g1)
        gss = (gs0, gs1)
        vbs = (v0, v1)
        sss = (ss0, ss1)

        zv = jnp.zeros((LANES,), jnp.float32)

        def zero_v(r16, c):
            for k in range(LANES):
                for f in range(feat // LANES):
                    v0[r16 * LANES + k, pl.ds(f * LANES, LANES)] = zv
            return c
        lax.fori_loop(0, CH // LANES, zero_v, 0)

        def zero_sh(j, c):
            pltpu.sync_copy(v0, s_sh.at[pl.ds(sid * rpt + j * CH, CH)])
            return c
        lax.fori_loop(0, rpt // CH, zero_sh, 0)
        plsc.subcore_barrier()

        def idx_start(j, b, sem):
            pltpu.async_copy(ebc_hbm.at[wid, j], ibs[b], sem)
            pltpu.async_copy(ewx_hbm.at[wid, j], ebs[b], sem)

        def idx_wait(j, b, sem):
            pltpu.make_async_copy(ebc_hbm.at[wid, j], ibs[b], sem).wait()
            pltpu.make_async_copy(ewx_hbm.at[wid, j], ebs[b], sem).wait()

        def gather_start(b2, b4, sem):
            pltpu.async_copy(xs_hbm.at[ibs[b4].at[0]], gbs[b2], sem)

        def gather_wait(b2, b4, sem):
            pltpu.make_async_copy(xs_hbm.at[ibs[b4].at[0]], gbs[b2],
                                  sem).wait()

        def scat_start(b2, b4, sem):
            pltpu.async_copy(vbs[b2], s_sh.at[ibs[b4].at[1]], sem, add=True)

        def scat_wait(b2, b4, sem):
            pltpu.make_async_copy(vbs[b2], s_sh.at[ibs[b4].at[1]], sem).wait()

        def scale(b2, b4):
            g = gbs[b2]
            v = vbs[b2]
            eb = ebs[b4]

            def inner(r2, cc):
                for u in range(2):
                    row = r2 * 2 + u
                    s16 = eb[pl.ds(row * LANES, LANES)]
                    for f in range(feat // LANES):
                        sl = pl.ds(f * LANES, LANES)
                        v[row, sl] = g[row, sl] * s16
                return cc
            lax.fori_loop(0, CH // 2, inner, 0)

        def step(j, b, has_prev2, has_next2):
            b2 = b % 2
            b4 = b % 4
            if has_prev2:
                scat_wait(b2, b4, sss[b2])
            if has_next2:
                idx_start(j + 2, (b + 2) % 4, iss[(b + 2) % 4])
            gather_wait(b2, b4, gss[b2])
            scale(b2, b4)
            scat_start(b2, b4, sss[b2])
            if has_next2:
                idx_wait(j + 2, (b + 2) % 4, iss[(b + 2) % 4])
                gather_start(b2, (b + 2) % 4, gss[b2])

        idx_start(0, 0, is0)
        idx_start(1, 1, is1)
        idx_wait(0, 0, is0)
        gather_start(0, 0, gs0)
        idx_wait(1, 1, is1)
        gather_start(1, 1, gs1)
        for b in range(4):
            step(b, b, b >= 2, True)

        def quad(t, c):
            j = 4 * t
            for b in range(4):
                step(j + b, b, True, True)
            return c
        lax.fori_loop(1, cpw // 4 - 1, quad, 0)

        jl = cpw - 4
        for b in range(4):
            step(jl + b, b, True, b < 2)
        scat_wait(0, 2, ss0)
        scat_wait(1, 3, ss1)

        plsc.subcore_barrier()
        pltpu.sync_copy(s_sh.at[pl.ds(sid * rpt, rpt)],
                        s_hbm.at[cid, pl.ds(sid * rpt, rpt)])

    return pl.kernel(
        body,
        out_type=jax.ShapeDtypeStruct((NC, np_, feat), jnp.float32),
        mesh=plsc.VectorSubcoreMesh(core_axis_name="c", subcore_axis_name="s"),
        scratch_types=[
            pltpu.VMEM((2, CH), jnp.int32),
            pltpu.VMEM((2, CH), jnp.int32),
            pltpu.VMEM((2, CH), jnp.int32),
            pltpu.VMEM((2, CH), jnp.int32),
            pltpu.VMEM((CH * LANES,), jnp.float32),
            pltpu.VMEM((CH * LANES,), jnp.float32),
            pltpu.VMEM((CH * LANES,), jnp.float32),
            pltpu.VMEM((CH * LANES,), jnp.float32),
            pltpu.VMEM((CH, feat), jnp.float32),
            pltpu.VMEM((CH, feat), jnp.float32),
            pltpu.VMEM((CH, feat), jnp.float32),
            pltpu.VMEM((CH, feat), jnp.float32),
            pltpu.VMEM_SHARED((np_, feat), jnp.float32),
            pltpu.SemaphoreType.DMA, pltpu.SemaphoreType.DMA,
            pltpu.SemaphoreType.DMA, pltpu.SemaphoreType.DMA,
            pltpu.SemaphoreType.DMA, pltpu.SemaphoreType.DMA,
            pltpu.SemaphoreType.DMA, pltpu.SemaphoreType.DMA,
        ],
    )


def _prep_body(deg_ref, x_ref, dinv_ref, xs_ref):
    d = 1.0 + deg_ref[0] + deg_ref[1]
    dinv = lax.rsqrt(d)
    dinv_ref[...] = dinv
    xs_ref[...] = x_ref[...] * dinv


def _dense_body(hid, s_ref, xs_ref, dinv_ref, h0_ref,
                wcz, wcr, wch, wlz, wlr, wlh, wout,
                bcz, bcr, bch, blz, blr, blh, bout,
                y_ref, h_ref, wz1, wr1, wh1, bz, br, bh):
    f32 = jnp.float32

    @pl.when(pl.program_id(0) == 0)
    def _():
        wz1[...] = jnp.dot(wcz[...], wlz[0:hid, :], preferred_element_type=f32)
        wr1[...] = jnp.dot(wcr[...], wlr[0:hid, :], preferred_element_type=f32)
        wh1[...] = jnp.dot(wch[...], wlh[0:hid, :], preferred_element_type=f32)
        bz[...] = jnp.dot(bcz[...], wlz[0:hid, :], preferred_element_type=f32) + blz[...]
        br[...] = jnp.dot(bcr[...], wlr[0:hid, :], preferred_element_type=f32) + blr[...]
        bh[...] = jnp.dot(bch[...], wlh[0:hid, :], preferred_element_type=f32) + blh[...]

    p = dinv_ref[...] * (s_ref[0] + s_ref[1] + xs_ref[...])
    h0 = h0_ref[...]
    zl = (jnp.dot(p, wz1[...], preferred_element_type=f32)
          + jnp.dot(h0, wlz[hid:2 * hid, :], preferred_element_type=f32) + bz[...])
    z = jax.nn.sigmoid(zl)
    rl = (jnp.dot(p, wr1[...], preferred_element_type=f32)
          + jnp.dot(h0, wlr[hid:2 * hid, :], preferred_element_type=f32) + br[...])
    r = jax.nn.sigmoid(rl)
    hl = (jnp.dot(p, wh1[...], preferred_element_type=f32)
          + jnp.dot(h0 * r, wlh[hid:2 * hid, :], preferred_element_type=f32) + bh[...])
    ht = jnp.tanh(hl)
    h = z * h0 + (1.0 - z) * ht
    h_ref[...] = h
    y_ref[...] = jnp.dot(jnp.maximum(h, 0.0), wout[...],
                         preferred_element_type=f32) + bout[...]


def kernel(g, node_feat, edge_weight, hidden_state,
           Wc_z, bc_z, Wc_r, bc_r, Wc_h, bc_h,
           Wl_z, bl_z, Wl_r, bl_r, Wl_h, bl_h, W_out, b_out):
    f32 = jnp.float32
    n, feat = node_feat.shape
    hid = hidden_state.shape[1]
    e = g.shape[1]

    npad = -(-n // (NS * LANES * 8)) * (NS * LANES * 8)
    cpw = -(-e // (NW * CH))
    cpw = -(-cpw // 8) * 8
    ep = cpw * CH * NW
    rpt = npad // NS

    src = jnp.concatenate([g[0], jnp.zeros((ep - e,), g.dtype)])
    dst = jnp.concatenate([g[1], jnp.zeros((ep - e,), g.dtype)])
    ew = jnp.concatenate([edge_weight, jnp.zeros((ep - e,), f32)])
    src3 = src.reshape(NW, cpw, CH)
    dst3 = dst.reshape(NW, cpw, CH)
    ew3 = ew.reshape(NW, cpw, CH)
    ebc = jnp.stack([src3, dst3], axis=2)
    ewx = jnp.broadcast_to(ew3[..., None],
                           (NW, cpw, CH, LANES)).reshape(NW, cpw, CH * LANES)
    x_pad = jnp.concatenate([node_feat, jnp.zeros((npad - n, feat), f32)])
    h0_pad = jnp.concatenate([hidden_state, jnp.zeros((npad - n, hid), f32)])

    deg_p = _deg_kernel(npad, cpw, rpt)(dst3, ew3)
    deg_col = deg_p.reshape(NC, npad, 1)

    nb = 10
    blk = npad // nb
    dinv, xs = pl.pallas_call(
        _prep_body,
        grid=(nb,),
        in_specs=[
            pl.BlockSpec((NC, blk, 1), lambda i: (0, i, 0)),
            pl.BlockSpec((blk, feat), lambda i: (i, 0)),
        ],
        out_specs=[
            pl.BlockSpec((blk, 1), lambda i: (i, 0)),
            pl.BlockSpec((blk, feat), lambda i: (i, 0)),
        ],
        out_shape=[
            jax.ShapeDtypeStruct((npad, 1), f32),
            jax.ShapeDtypeStruct((npad, feat), f32),
        ],
    )(deg_col, x_pad)

    s_p = _scatter_kernel(npad, feat, cpw, rpt)(ebc, ewx, xs)

    full = lambda shape: pl.BlockSpec(shape, lambda i: tuple(0 for _ in shape))
    y_pad, h_pad = pl.pallas_call(
        functools.partial(_dense_body, hid),
        grid=(nb,),
        in_specs=[
            pl.BlockSpec((NC, blk, feat), lambda i: (0, i, 0)),
            pl.BlockSpec((blk, feat), lambda i: (i, 0)),
            pl.BlockSpec((blk, 1), lambda i: (i, 0)),
            pl.BlockSpec((blk, hid), lambda i: (i, 0)),
            full((feat, hid)), full((feat, hid)), full((feat, hid)),
            full((2 * hid, hid)), full((2 * hid, hid)), full((2 * hid, hid)),
            full((hid, feat)),
            full((1, hid)), full((1, hid)), full((1, hid)),
            full((1, hid)), full((1, hid)), full((1, hid)),
            full((1, feat)),
        ],
        out_specs=[
            pl.BlockSpec((blk, feat), lambda i: (i, 0)),
            pl.BlockSpec((blk, hid), lambda i: (i, 0)),
        ],
        out_shape=[
            jax.ShapeDtypeStruct((npad, feat), f32),
            jax.ShapeDtypeStruct((npad, hid), f32),
        ],
        scratch_shapes=[
            pltpu.VMEM((feat, hid), f32), pltpu.VMEM((feat, hid), f32),
            pltpu.VMEM((feat, hid), f32),
            pltpu.VMEM((1, hid), f32), pltpu.VMEM((1, hid), f32),
            pltpu.VMEM((1, hid), f32),
        ],
    )(s_p, xs, dinv, h0_pad,
      Wc_z, Wc_r, Wc_h, Wl_z, Wl_r, Wl_h, W_out,
      bc_z.reshape(1, hid), bc_r.reshape(1, hid), bc_h.reshape(1, hid),
      bl_z.reshape(1, hid), bl_r.reshape(1, hid), bl_h.reshape(1, hid),
      b_out.reshape(1, feat))

    return y_pad[:n], h_pad[:n]


# speedup vs baseline: 1.1628x; 1.1628x over previous
import functools

import jax
import jax.numpy as jnp
from jax import lax
from jax.experimental import pallas as pl
from jax.experimental.pallas import tpu as pltpu
from jax.experimental.pallas import tpu_sc as plsc

NC = 2
NS = 16
NW = NC * NS
CH = 64
LANES = 16


def _deg_kernel(np_, cpw, rpt):
    batch_k = 8

    def body(dst_hbm, ew_hbm, deg_hbm, dbuf, ebuf, zbuf, deg_sh, sem):
        cid = lax.axis_index("c")
        sid = lax.axis_index("s")
        wid = cid * NS + sid
        zv = jnp.zeros((LANES,), jnp.float32)

        def zero_buf(j, c):
            zbuf[pl.ds(j * LANES, LANES)] = zv
            return c
        lax.fori_loop(0, rpt // LANES, zero_buf, 0)
        pltpu.sync_copy(zbuf, deg_sh.at[pl.ds(sid * rpt, rpt)])
        pltpu.sync_copy(dst_hbm.at[wid], dbuf)
        pltpu.sync_copy(ew_hbm.at[wid], ebuf)
        plsc.subcore_barrier()

        def batch(t, c):
            for b in range(batch_k):
                j = t * batch_k + b
                pltpu.async_copy(ebuf.at[j], deg_sh.at[dbuf.at[j]], sem,
                                 add=True)
            for b in range(batch_k):
                j = t * batch_k + b
                pltpu.make_async_copy(ebuf.at[j], deg_sh.at[dbuf.at[j]],
                                      sem).wait()
            return c
        lax.fori_loop(0, cpw // batch_k, batch, 0)
        plsc.subcore_barrier()
        pltpu.sync_copy(deg_sh.at[pl.ds(sid * rpt, rpt)],
                        deg_hbm.at[cid, pl.ds(sid * rpt, rpt)])

    return pl.kernel(
        body,
        out_type=jax.ShapeDtypeStruct((NC, np_), jnp.float32),
        mesh=plsc.VectorSubcoreMesh(core_axis_name="c", subcore_axis_name="s"),
        scratch_types=[
            pltpu.VMEM((cpw, CH), jnp.int32),
            pltpu.VMEM((cpw, CH), jnp.float32),
            pltpu.VMEM((rpt,), jnp.float32),
            pltpu.VMEM_SHARED((np_,), jnp.float32),
            pltpu.SemaphoreType.DMA,
        ],
    )


def _scatter_kernel(np_, feat, cpw, rpt):
    def body(ebc_hbm, ewx_hbm, xs_hbm, s_hbm,
             ib0, ib1, ib2, ib3, eb0, eb1, eb2, eb3,
             g0, g1, v0, v1, s_sh,
             is0, is1, is2, is3, gs0, gs1, ss0, ss1):
        cid = lax.axis_index("c")
        sid = lax.axis_index("s")
        wid = cid * NS + sid
        ibs = (ib0, ib1, ib2, ib3)
        ebs = (eb0, eb1, eb2, eb3)
        iss = (is0, is1, is2, is3)
        gbs = (g0, g1)
        gss = (gs0, gs1)
        vbs = (v0, v1)
        sss = (ss0, ss1)

        zv = jnp.zeros((LANES,), jnp.float32)

        def zero_v(r16, c):
            for k in range(LANES):
                for f in range(feat // LANES):
                    v0[r16 * LANES + k, pl.ds(f * LANES, LANES)] = zv
            return c
        lax.fori_loop(0, CH // LANES, zero_v, 0)

        def zero_sh(j, c):
            pltpu.sync_copy(v0, s_sh.at[pl.ds(sid * rpt + j * CH, CH)])
            return c
        lax.fori_loop(0, rpt // CH, zero_sh, 0)
        plsc.subcore_barrier()

        def idx_start(j, b, sem):
            pltpu.async_copy(ebc_hbm.at[wid, j], ibs[b], sem)
            pltpu.async_copy(ewx_hbm.at[wid, j], ebs[b], sem)

        def idx_wait(j, b, sem):
            pltpu.make_async_copy(ebc_hbm.at[wid, j], ibs[b], sem).wait()
            pltpu.make_async_copy(ewx_hbm.at[wid, j], ebs[b], sem).wait()
            off = jnp.full((LANES,), cid * np_, jnp.int32)
            for q in range(CH // LANES):
                sl = pl.ds(q * LANES, LANES)
                ibs[b][0, sl] = ibs[b][0, sl] + off

        def gather_start(b2, b4, sem):
            pltpu.async_copy(xs_hbm.at[ibs[b4].at[0]], gbs[b2], sem)

        def gather_wait(b2, b4, sem):
            pltpu.make_async_copy(xs_hbm.at[ibs[b4].at[0]], gbs[b2],
                                  sem).wait()

        def scat_start(b2, b4, sem):
            pltpu.async_copy(vbs[b2], s_sh.at[ibs[b4].at[1]], sem, add=True)

        def scat_wait(b2, b4, sem):
            pltpu.make_async_copy(vbs[b2], s_sh.at[ibs[b4].at[1]], sem).wait()

        def scale(b2, b4):
            g = gbs[b2]
            v = vbs[b2]
            eb = ebs[b4]

            def inner(r2, cc):
                for u in range(2):
                    row = r2 * 2 + u
                    s16 = eb[pl.ds(row * LANES, LANES)]
                    for f in range(feat // LANES):
                        sl = pl.ds(f * LANES, LANES)
                        v[row, sl] = g[row, sl] * s16
                return cc
            lax.fori_loop(0, CH // 2, inner, 0)

        def step(j, b, has_prev2, has_next2):
            b2 = b % 2
            b4 = b % 4
            if has_prev2:
                scat_wait(b2, b4, sss[b2])
            if has_next2:
                idx_start(j + 2, (b + 2) % 4, iss[(b + 2) % 4])
            gather_wait(b2, b4, gss[b2])
            scale(b2, b4)
            scat_start(b2, b4, sss[b2])
            if has_next2:
                idx_wait(j + 2, (b + 2) % 4, iss[(b + 2) % 4])
                gather_start(b2, (b + 2) % 4, gss[b2])

        idx_start(0, 0, is0)
        idx_start(1, 1, is1)
        idx_wait(0, 0, is0)
        gather_start(0, 0, gs0)
        idx_wait(1, 1, is1)
        gather_start(1, 1, gs1)
        for b in range(4):
            step(b, b, b >= 2, True)

        def quad(t, c):
            j = 4 * t
            for b in range(4):
                step(j + b, b, True, True)
            return c
        lax.fori_loop(1, cpw // 4 - 1, quad, 0)

        jl = cpw - 4
        for b in range(4):
            step(jl + b, b, True, b < 2)
        scat_wait(0, 2, ss0)
        scat_wait(1, 3, ss1)

        plsc.subcore_barrier()
        pltpu.sync_copy(s_sh.at[pl.ds(sid * rpt, rpt)],
                        s_hbm.at[cid, pl.ds(sid * rpt, rpt)])

    return pl.kernel(
        body,
        out_type=jax.ShapeDtypeStruct((NC, np_, feat), jnp.float32),
        mesh=plsc.VectorSubcoreMesh(core_axis_name="c", subcore_axis_name="s"),
        scratch_types=[
            pltpu.VMEM((2, CH), jnp.int32),
            pltpu.VMEM((2, CH), jnp.int32),
            pltpu.VMEM((2, CH), jnp.int32),
            pltpu.VMEM((2, CH), jnp.int32),
            pltpu.VMEM((CH * LANES,), jnp.float32),
            pltpu.VMEM((CH * LANES,), jnp.float32),
            pltpu.VMEM((CH * LANES,), jnp.float32),
            pltpu.VMEM((CH * LANES,), jnp.float32),
            pltpu.VMEM((CH, feat), jnp.float32),
            pltpu.VMEM((CH, feat), jnp.float32),
            pltpu.VMEM((CH, feat), jnp.float32),
            pltpu.VMEM((CH, feat), jnp.float32),
            pltpu.VMEM_SHARED((np_, feat), jnp.float32),
            pltpu.SemaphoreType.DMA, pltpu.SemaphoreType.DMA,
            pltpu.SemaphoreType.DMA, pltpu.SemaphoreType.DMA,
            pltpu.SemaphoreType.DMA, pltpu.SemaphoreType.DMA,
            pltpu.SemaphoreType.DMA, pltpu.SemaphoreType.DMA,
        ],
    )


def _prep_body(deg_ref, x_ref, dinv_ref, xs_ref):
    d = 1.0 + deg_ref[0] + deg_ref[1]
    dinv = lax.rsqrt(d)
    dinv_ref[...] = dinv
    xs_ref[...] = x_ref[...] * dinv


def _dense_body(hid, s_ref, xs_ref, dinv_ref, h0_ref,
                wcz, wcr, wch, wlz, wlr, wlh, wout,
                bcz, bcr, bch, blz, blr, blh, bout,
                y_ref, h_ref, wz1, wr1, wh1, bz, br, bh):
    f32 = jnp.float32

    @pl.when(pl.program_id(0) == 0)
    def _():
        wz1[...] = jnp.dot(wcz[...], wlz[0:hid, :], preferred_element_type=f32)
        wr1[...] = jnp.dot(wcr[...], wlr[0:hid, :], preferred_element_type=f32)
        wh1[...] = jnp.dot(wch[...], wlh[0:hid, :], preferred_element_type=f32)
        bz[...] = jnp.dot(bcz[...], wlz[0:hid, :], preferred_element_type=f32) + blz[...]
        br[...] = jnp.dot(bcr[...], wlr[0:hid, :], preferred_element_type=f32) + blr[...]
        bh[...] = jnp.dot(bch[...], wlh[0:hid, :], preferred_element_type=f32) + blh[...]

    p = dinv_ref[...] * (s_ref[0] + s_ref[1] + xs_ref[...])
    h0 = h0_ref[...]
    zl = (jnp.dot(p, wz1[...], preferred_element_type=f32)
          + jnp.dot(h0, wlz[hid:2 * hid, :], preferred_element_type=f32) + bz[...])
    z = jax.nn.sigmoid(zl)
    rl = (jnp.dot(p, wr1[...], preferred_element_type=f32)
          + jnp.dot(h0, wlr[hid:2 * hid, :], preferred_element_type=f32) + br[...])
    r = jax.nn.sigmoid(rl)
    hl = (jnp.dot(p, wh1[...], preferred_element_type=f32)
          + jnp.dot(h0 * r, wlh[hid:2 * hid, :], preferred_element_type=f32) + bh[...])
    ht = jnp.tanh(hl)
    h = z * h0 + (1.0 - z) * ht
    h_ref[...] = h
    y_ref[...] = jnp.dot(jnp.maximum(h, 0.0), wout[...],
                         preferred_element_type=f32) + bout[...]


def kernel(g, node_feat, edge_weight, hidden_state,
           Wc_z, bc_z, Wc_r, bc_r, Wc_h, bc_h,
           Wl_z, bl_z, Wl_r, bl_r, Wl_h, bl_h, W_out, b_out):
    f32 = jnp.float32
    n, feat = node_feat.shape
    hid = hidden_state.shape[1]
    e = g.shape[1]

    npad = -(-n // (NS * LANES * 8)) * (NS * LANES * 8)
    cpw = -(-e // (NW * CH))
    cpw = -(-cpw // 8) * 8
    ep = cpw * CH * NW
    rpt = npad // NS

    src = jnp.concatenate([g[0], jnp.zeros((ep - e,), g.dtype)])
    dst = jnp.concatenate([g[1], jnp.zeros((ep - e,), g.dtype)])
    ew = jnp.concatenate([edge_weight, jnp.zeros((ep - e,), f32)])
    src3 = src.reshape(NW, cpw, CH)
    dst3 = dst.reshape(NW, cpw, CH)
    ew3 = ew.reshape(NW, cpw, CH)
    ebc = jnp.stack([src3, dst3], axis=2)
    ewx = jnp.broadcast_to(ew3[..., None],
                           (NW, cpw, CH, LANES)).reshape(NW, cpw, CH * LANES)
    x_pad = jnp.concatenate([node_feat, jnp.zeros((npad - n, feat), f32)])
    h0_pad = jnp.concatenate([hidden_state, jnp.zeros((npad - n, hid), f32)])

    deg_p = _deg_kernel(npad, cpw, rpt)(dst3, ew3)
    deg_col = deg_p.reshape(NC, npad, 1)

    nb = 10
    blk = npad // nb
    dinv, xs = pl.pallas_call(
        _prep_body,
        grid=(nb,),
        in_specs=[
            pl.BlockSpec((NC, blk, 1), lambda i: (0, i, 0)),
            pl.BlockSpec((blk, feat), lambda i: (i, 0)),
        ],
        out_specs=[
            pl.BlockSpec((blk, 1), lambda i: (i, 0)),
            pl.BlockSpec((blk, feat), lambda i: (i, 0)),
        ],
        out_shape=[
            jax.ShapeDtypeStruct((npad, 1), f32),
            jax.ShapeDtypeStruct((npad, feat), f32),
        ],
    )(deg_col, x_pad)

    xs2 = jnp.concatenate([xs, xs], axis=0)
    s_p = _scatter_kernel(npad, feat, cpw, rpt)(ebc, ewx, xs2)

    full = lambda shape: pl.BlockSpec(shape, lambda i: tuple(0 for _ in shape))
    y_pad, h_pad = pl.pallas_call(
        functools.partial(_dense_body, hid),
        grid=(nb,),
        in_specs=[
            pl.BlockSpec((NC, blk, feat), lambda i: (0, i, 0)),
            pl.BlockSpec((blk, feat), lambda i: (i, 0)),
            pl.BlockSpec((blk, 1), lambda i: (i, 0)),
            pl.BlockSpec((blk, hid), lambda i: (i, 0)),
            full((feat, hid)), full((feat, hid)), full((feat, hid)),
            full((2 * hid, hid)), full((2 * hid, hid)), full((2 * hid, hid)),
            full((hid, feat)),
            full((1, hid)), full((1, hid)), full((1, hid)),
            full((1, hid)), full((1, hid)), full((1, hid)),
            full((1, feat)),
        ],
        out_specs=[
            pl.BlockSpec((blk, feat), lambda i: (i, 0)),
            pl.BlockSpec((blk, hid), lambda i: (i, 0)),
        ],
        out_shape=[
            jax.ShapeDtypeStruct((npad, feat), f32),
            jax.ShapeDtypeStruct((npad, hid), f32),
        ],
        scratch_shapes=[
            pltpu.VMEM((feat, hid), f32), pltpu.VMEM((feat, hid), f32),
            pltpu.VMEM((feat, hid), f32),
            pltpu.VMEM((1, hid), f32), pltpu.VMEM((1, hid), f32),
            pltpu.VMEM((1, hid), f32),
        ],
    )(s_p, xs, dinv, h0_pad,
      Wc_z, Wc_r, Wc_h, Wl_z, Wl_r, Wl_h, W_out,
      bc_z.reshape(1, hid), bc_r.reshape(1, hid), bc_h.reshape(1, hid),
      bl_z.reshape(1, hid), bl_r.reshape(1, hid), bl_h.reshape(1, hid),
      b_out.reshape(1, feat))

    return y_pad[:n], h_pad[:n]
